# baseline (speedup 1.0000x reference)
import functools

import jax
import jax.numpy as jnp
from jax import lax
from jax.experimental import pallas as pl
from jax.experimental.pallas import tpu as pltpu
from jax.experimental.pallas import tpu_sc as plsc

N = 10000
E = 160000
H = 128
NRBF = 128
L = 2
T = 2
RC = 5.0
NG = 16

NC = 2
NS = 16
NW = NC * NS
CH = 64
EPAD = 163840
EW = EPAD // NW
NCHUNK = EW // CH
CHP = 128
NPRE = EW // CHP
NPAD = 10240
NSUB = NPAD // NS
TRASH = N
BE = 2048
BN = 1000


def _sc_mesh():
    return plsc.VectorSubcoreMesh(core_axis_name="c", subcore_axis_name="s",
                                  num_cores=NC, num_subcores=NS)


def _pre_body(posx_h, posy_h, posz_h, an_h, j_h, i_h, d2_h, idx_h, te_h,
              jv, iv, xj, yj, zj, xi, yi, zi, tv, d2v, idxv, tev, sem):
    w = lax.axis_index("s") * NC + lax.axis_index("c")
    base = w * EW
    pltpu.sync_copy(j_h.at[pl.ds(base, EW)], jv)
    pltpu.sync_copy(i_h.at[pl.ds(base, EW)], iv)

    def chunk(c, carry):
        s = c * CHP
        jc = jv.at[pl.ds(s, CHP)]
        ic = iv.at[pl.ds(s, CHP)]
        ds = pl.ds(s, CHP)
        descs = [
            pltpu.async_copy(posx_h.at[jc], xj.at[ds], sem),
            pltpu.async_copy(posy_h.at[jc], yj.at[ds], sem),
            pltpu.async_copy(posz_h.at[jc], zj.at[ds], sem),
            pltpu.async_copy(posx_h.at[ic], xi.at[ds], sem),
            pltpu.async_copy(posy_h.at[ic], yi.at[ds], sem),
            pltpu.async_copy(posz_h.at[ic], zi.at[ds], sem),
            pltpu.async_copy(an_h.at[ic], tv.at[ds], sem),
        ]
        for d in descs:
            d.wait()
        return carry

    lax.fori_loop(0, NPRE, chunk, 0)

    def q_body(q, carry):
        s = q * 16
        sl = pl.ds(s, 16)
        dx = xj[sl] - xi[sl]
        dy = yj[sl] - yi[sl]
        dz = zj[sl] - zi[sl]
        d2v[sl] = dx * dx + dy * dy + dz * dz
        t16 = tv[sl]
        idxv[sl] = t16 * N + jv[sl]
        tev[sl] = t16
        return carry

    lax.fori_loop(0, EW // 16, q_body, 0)
    pltpu.sync_copy(d2v, d2_h.at[pl.ds(base, EW)])
    pltpu.sync_copy(idxv, idx_h.at[pl.ds(base, EW)])
    pltpu.sync_copy(tev, te_h.at[pl.ds(base, EW)])


def _pre(posx, posy, posz, an, jp, ip):
    k = pl.kernel(
        _pre_body,
        out_type=[jax.ShapeDtypeStruct((EPAD,), jnp.float32),
                  jax.ShapeDtypeStruct((EPAD,), jnp.int32),
                  jax.ShapeDtypeStruct((EPAD,), jnp.int32)],
        mesh=_sc_mesh(),
        scratch_types=[pltpu.VMEM((EW,), jnp.int32),
                       pltpu.VMEM((EW,), jnp.int32),
                       pltpu.VMEM((EW,), jnp.float32),
                       pltpu.VMEM((EW,), jnp.float32),
                       pltpu.VMEM((EW,), jnp.float32),
                       pltpu.VMEM((EW,), jnp.float32),
                       pltpu.VMEM((EW,), jnp.float32),
                       pltpu.VMEM((EW,), jnp.float32),
                       pltpu.VMEM((EW,), jnp.int32),
                       pltpu.VMEM((EW,), jnp.float32),
                       pltpu.VMEM((EW,), jnp.int32),
                       pltpu.VMEM((EW,), jnp.int32),
                       pltpu.SemaphoreType.DMA],
    )
    return k(posx, posy, posz, an, jp, ip)


def _gs_body(tab_h, filt_h, gidx_h, sidx_h, zeros_h, out_h,
             gidx_v, sidx_v, pa, pb, fa, fb, acc,
             sem_pa, sem_pb, sem_fa, sem_fb):
    cid = lax.axis_index("c")
    sid = lax.axis_index("s")
    w = sid * NC + cid
    base = w * NCHUNK
    r0 = sid * NSUB
    pltpu.sync_copy(zeros_h.at[pl.ds(r0, NSUB)], acc.at[pl.ds(r0, NSUB)])
    pltpu.sync_copy(gidx_h.at[pl.ds(w * EW, EW)], gidx_v)
    pltpu.sync_copy(sidx_h.at[pl.ds(base, NCHUNK)], sidx_v)
    plsc.subcore_barrier()

    def fire(c, pbuf, fbuf, psem, fsem):
        pltpu.async_copy(tab_h.at[gidx_v.at[pl.ds(c * CH, CH)]], pbuf, psem)
        pltpu.async_copy(filt_h.at[pl.ds((base + c) * CH, CH)], fbuf, fsem)

    def drain(c, pbuf, fbuf, psem, fsem):
        pltpu.make_async_copy(tab_h.at[gidx_v.at[pl.ds(0, CH)]], pbuf,
                              psem).wait()
        pltpu.make_async_copy(filt_h.at[pl.ds(base * CH, CH)], fbuf,
                              fsem).wait()

    def mul_scatter(c, pbuf, fbuf):
        def row(r, carry):
            for h in range(H // 16):
                sl = pl.ds(h * 16, 16)
                pbuf[r, sl] = pbuf[r, sl] * fbuf[r, sl]
            return carry

        lax.fori_loop(0, CH, row, 0)
        pltpu.sync_copy(pbuf, acc.at[sidx_v.at[c]], add=True)

    fire(0, pa, fa, sem_pa, sem_fa)

    def k_body(k, carry):
        c = 2 * k
        fire(c + 1, pb, fb, sem_pb, sem_fb)
        drain(c, pa, fa, sem_pa, sem_fa)
        mul_scatter(c, pa, fa)

        @pl.when(c + 2 < NCHUNK)
        def _():
            fire(c + 2, pa, fa, sem_pa, sem_fa)

        drain(c + 1, pb, fb, sem_pb, sem_fb)
        mul_scatter(c + 1, pb, fb)
        return carry

    lax.fori_loop(0, NCHUNK // 2, k_body, 0)
    plsc.subcore_barrier()
    pltpu.sync_copy(acc.at[pl.ds(r0, NSUB)], out_h.at[cid, pl.ds(r0, NSUB)])


def _gs(tab, filt, gidx2d, sidx2d, zerosN):
    buf = pltpu.VMEM((CH, H), jnp.float32)
    k = pl.kernel(
        _gs_body,
        out_type=jax.ShapeDtypeStruct((NC, NPAD, H), jnp.float32),
        mesh=_sc_mesh(),
        scratch_types=[pltpu.VMEM((EW,), jnp.int32),
                       pltpu.VMEM((NCHUNK, CH), jnp.int32),
                       buf, buf, buf, buf,
                       pltpu.VMEM_SHARED((NPAD, H), jnp.float32),
                       pltpu.SemaphoreType.DMA, pltpu.SemaphoreType.DMA,
                       pltpu.SemaphoreType.DMA, pltpu.SemaphoreType.DMA],
    )
    return k(tab, filt, gidx2d, sidx2d, zerosN)


def _phi0_kernel(an_ref, emb_ref, w10, b10, w20, b20, w11, b11, w21, b21,
                 out_ref):
    a = an_ref[0, 0, :]
    x = jnp.where(a[:, None] == 0, emb_ref[0:1, :], emb_ref[1:2, :])
    _phi_core(x, w10, b10, w20, b20, w11, b11, w21, b21, out_ref)


def _phi0(an3, embed, w10, b10, w20, b20, w11, b11, w21, b21):
    nb = N // BN
    wspec = pl.BlockSpec((H, H), lambda b: (0, 0))
    bspec = pl.BlockSpec((1, H), lambda b: (0, 0))
    return pl.pallas_call(
        _phi0_kernel,
        grid=(nb,),
        in_specs=[pl.BlockSpec((1, 1, BN), lambda b: (b, 0, 0)),
                  pl.BlockSpec((T, H), lambda b: (0, 0)),
                  wspec, bspec, wspec, bspec, wspec, bspec, wspec, bspec],
        out_specs=pl.BlockSpec((2, BN, H), lambda b: (0, b, 0)),
        out_shape=jax.ShapeDtypeStruct((2, N, H), jnp.float32),
    )(an3, embed, w10, b10, w20, b20, w11, b11, w21, b21)


def _phi_core(x, w10, b10, w20, b20, w11, b11, w21, b21, out_ref):
    h0 = jax.nn.silu(jnp.dot(x, w10[...], preferred_element_type=jnp.float32)
                     + b10[...])
    p0 = jnp.dot(h0, w20[...], preferred_element_type=jnp.float32) + b20[...]
    h1 = jax.nn.silu(jnp.dot(x, w11[...], preferred_element_type=jnp.float32)
                     + b11[...])
    p1 = jnp.dot(h1, w21[...], preferred_element_type=jnp.float32) + b21[...]
    out_ref[0, :, :] = p0
    out_ref[1, :, :] = p1


def _phi_kernel(xp_ref, w10, b10, w20, b20, w11, b11, w21, b21, out_ref):
    x = xp_ref[0, :, :] + xp_ref[1, :, :]
    _phi_core(x, w10, b10, w20, b20, w11, b11, w21, b21, out_ref)


def _phi(xp, w10, b10, w20, b20, w11, b11, w21, b21):
    nb = N // BN
    wspec = pl.BlockSpec((H, H), lambda b: (0, 0))
    bspec = pl.BlockSpec((1, H), lambda b: (0, 0))
    return pl.pallas_call(
        _phi_kernel,
        grid=(nb,),
        in_specs=[pl.BlockSpec((2, BN, H), lambda b: (0, b, 0)),
                  wspec, bspec, wspec, bspec, wspec, bspec, wspec, bspec],
        out_specs=pl.BlockSpec((2, BN, H), lambda b: (0, b, 0)),
        out_shape=jax.ShapeDtypeStruct((2, N, H), jnp.float32),
    )(xp, w10, b10, w20, b20, w11, b11, w21, b21)


def _filt_kernel(d2_ref, te_ref, wf00, bf00, wf01, bf01,
                 wf10, bf10, wf11, bf11, out0_ref, out1_ref):
    d2 = d2_ref[0, 0, :]
    d = jnp.sqrt(d2)
    d = jnp.where(d <= 1e-6, 1e-6, d)
    u = d * (1.0 / RC)
    u2 = u * u
    u4 = u2 * u2
    u5 = u4 * u
    env = 1.0 - 21.0 * u5 + 35.0 * u5 * u - 15.0 * u5 * u2
    env = jnp.where(u < 1.0, env, 0.0)
    offs = lax.broadcasted_iota(jnp.int32, (1, NRBF), 1).astype(jnp.float32) * (
        1.0 / (NRBF - 1))
    delta = 1.0 / (NRBF - 1)
    coeff = -0.5 / (delta * delta)
    diff = u[:, None] - offs
    rbf = jnp.exp(coeff * (diff * diff)) * env[:, None]
    t = te_ref[0, 0, :]
    m = t[:, None] == 0
    f00 = jnp.dot(rbf, wf00[...], preferred_element_type=jnp.float32) + bf00[...]
    f01 = jnp.dot(rbf, wf01[...], preferred_element_type=jnp.float32) + bf01[...]
    out0_ref[...] = jnp.where(m, f00, f01)
    f10 = jnp.dot(rbf, wf10[...], preferred_element_type=jnp.float32) + bf10[...]
    f11 = jnp.dot(rbf, wf11[...], preferred_element_type=jnp.float32) + bf11[...]
    out1_ref[...] = jnp.where(m, f10, f11)


def _filt2(d2r, ter, wf00, bf00, wf01, bf01, wf10, bf10, wf11, bf11):
    nbe = EPAD // BE
    wspec = pl.BlockSpec((NRBF, H), lambda b: (0, 0))
    bspec = pl.BlockSpec((1, H), lambda b: (0, 0))
    espec = pl.BlockSpec((BE, H), lambda b: (b, 0))
    eshape = jax.ShapeDtypeStruct((EPAD, H), jnp.float32)
    return pl.pallas_call(
        _filt_kernel,
        grid=(nbe,),
        in_specs=[pl.BlockSpec((1, 1, BE), lambda b: (b, 0, 0)),
                  pl.BlockSpec((1, 1, BE), lambda b: (b, 0, 0)),
                  wspec, bspec, wspec, bspec, wspec, bspec, wspec, bspec],
        out_specs=[espec, espec],
        out_shape=[eshape, eshape],
    )(d2r, ter, wf00, bf00, wf01, bf01, wf10, bf10, wf11, bf11)


def _out_kernel(xp_ref, w1_ref, b1_ref, w2_ref, b2_ref, batch_ref, out_ref):
    b = pl.program_id(0)
    x = xp_ref[0, :, :] + xp_ref[1, :, :]
    h = jax.nn.silu(jnp.dot(x, w1_ref[...], preferred_element_type=jnp.float32)
                    + b1_ref[...]) * (1.0 / 0.6)
    pa = jnp.dot(h, w2_ref[...], preferred_element_type=jnp.float32) + b2_ref[...]
    bt = batch_ref[0, 0, :]
    oh = (bt[:, None] == lax.broadcasted_iota(jnp.int32, (1, NG), 1)
          ).astype(jnp.float32)
    partial = lax.dot_general(pa, oh, (((0,), (0,)), ((), ())))

    @pl.when(b == 0)
    def _():
        out_ref[...] = partial

    @pl.when(b != 0)
    def _():
        out_ref[...] = out_ref[...] + partial


def _out_stage(xp, Wo1, bo1, Wo2, bo2, batch3):
    nb = N // BN
    out = pl.pallas_call(
        _out_kernel,
        grid=(nb,),
        in_specs=[
            pl.BlockSpec((2, BN, H), lambda b: (0, b, 0)),
            pl.BlockSpec((H, H // 2), lambda b: (0, 0)),
            pl.BlockSpec((1, H // 2), lambda b: (0, 0)),
            pl.BlockSpec((H // 2, 1), lambda b: (0, 0)),
            pl.BlockSpec((1, 1), lambda b: (0, 0)),
            pl.BlockSpec((1, 1, BN), lambda b: (b, 0, 0)),
        ],
        out_specs=pl.BlockSpec((1, NG), lambda b: (0, 0)),
        out_shape=jax.ShapeDtypeStruct((1, NG), jnp.float32),
    )(xp, Wo1, bo1.reshape(1, -1), Wo2, bo2.reshape(1, 1), batch3)
    return out.reshape(NG)


def kernel(pos, embed, Wm1, bm1, Wm2, bm2, Wf, bf, Wo1, bo1, Wo2, bo2,
           atomic_number, edge_index, batch):
    f32 = jnp.float32
    pos = pos.astype(f32)
    posx = pos[:, 0]
    posy = pos[:, 1]
    posz = pos[:, 2]
    an = atomic_number.astype(jnp.int32)
    j = edge_index[0].astype(jnp.int32)
    i = edge_index[1].astype(jnp.int32)
    pad = EPAD - E
    jp = jnp.concatenate([j, jnp.zeros((pad,), jnp.int32)])
    ip_g = jnp.concatenate([i, jnp.zeros((pad,), jnp.int32)])
    ip_s = jnp.concatenate([i, jnp.full((pad,), TRASH, jnp.int32)])

    d2, idx_phi, te = _pre(posx, posy, posz, an, jp, ip_g)

    an3 = an.reshape(N // BN, 1, BN)
    d2r = d2.reshape(EPAD // BE, 1, BE)
    ter = te.reshape(EPAD // BE, 1, BE)
    iscat = ip_s.reshape(EPAD // CH, CH)
    zerosN = jnp.zeros((NPAD, H), f32)
    batch3 = batch.astype(jnp.int32).reshape(N // BN, 1, BN)

    def mw(l):
        return (Wm1[l, 0], bm1[l, 0].reshape(1, H),
                Wm2[l, 0, :, :H], bm2[l, 0, :H].reshape(1, H),
                Wm1[l, 1], bm1[l, 1].reshape(1, H),
                Wm2[l, 1, :, :H], bm2[l, 1, :H].reshape(1, H))

    filt0, filt1 = _filt2(d2r, ter,
                          Wf[0, 0, :, :H], bf[0, 0, :H].reshape(1, H),
                          Wf[0, 1, :, :H], bf[0, 1, :H].reshape(1, H),
                          Wf[1, 0, :, :H], bf[1, 0, :H].reshape(1, H),
                          Wf[1, 1, :, :H], bf[1, 1, :H].reshape(1, H))
    phitab = _phi0(an3, embed.astype(f32), *mw(0))
    xp = _gs(phitab.reshape(2 * N, H), filt0, idx_phi, iscat, zerosN)
    phitab = _phi(xp, *mw(1))
    xp = _gs(phitab.reshape(2 * N, H), filt1, idx_phi, iscat, zerosN)

    return _out_stage(xp, Wo1, bo1, Wo2, bo2, batch3)


# speedup vs baseline: 1.0107x; 1.0107x over previous
import functools

import jax
import jax.numpy as jnp
from jax import lax
from jax.experimental import pallas as pl
from jax.experimental.pallas import tpu as pltpu
from jax.experimental.pallas import tpu_sc as plsc

N = 10000
E = 160000
H = 128
NRBF = 128
L = 2
T = 2
RC = 5.0
NG = 16

NC = 2
NS = 16
NW = NC * NS
CH = 64
EPAD = 163840
EW = EPAD // NW
NCHUNK = EW // CH
CHP = 128
NPRE = EW // CHP
NPAD = 10240
NSUB = NPAD // NS
TRASH = N
BE = 2048
BN = 1000


def _sc_mesh():
    return plsc.VectorSubcoreMesh(core_axis_name="c", subcore_axis_name="s",
                                  num_cores=NC, num_subcores=NS)


def _pre_body(posx_h, posy_h, posz_h, an_h, j_h, i_h, d2_h, idx_h, te_h,
              jv, iv, xj, yj, zj, xi, yi, zi, tv, d2v, idxv, tev, sem):
    w = lax.axis_index("s") * NC + lax.axis_index("c")
    base = w * EW
    pltpu.sync_copy(j_h.at[pl.ds(base, EW)], jv)
    pltpu.sync_copy(i_h.at[pl.ds(base, EW)], iv)

    def chunk(c, carry):
        s = c * CHP
        jc = jv.at[pl.ds(s, CHP)]
        ic = iv.at[pl.ds(s, CHP)]
        ds = pl.ds(s, CHP)
        descs = [
            pltpu.async_copy(posx_h.at[jc], xj.at[ds], sem),
            pltpu.async_copy(posy_h.at[jc], yj.at[ds], sem),
            pltpu.async_copy(posz_h.at[jc], zj.at[ds], sem),
            pltpu.async_copy(posx_h.at[ic], xi.at[ds], sem),
            pltpu.async_copy(posy_h.at[ic], yi.at[ds], sem),
            pltpu.async_copy(posz_h.at[ic], zi.at[ds], sem),
            pltpu.async_copy(an_h.at[ic], tv.at[ds], sem),
        ]
        for d in descs:
            d.wait()
        return carry

    lax.fori_loop(0, NPRE, chunk, 0)

    def q_body(q, carry):
        s = q * 16
        sl = pl.ds(s, 16)
        dx = xj[sl] - xi[sl]
        dy = yj[sl] - yi[sl]
        dz = zj[sl] - zi[sl]
        d2v[sl] = dx * dx + dy * dy + dz * dz
        t16 = tv[sl]
        idxv[sl] = t16 * N + jv[sl]
        tev[sl] = t16
        return carry

    lax.fori_loop(0, EW // 16, q_body, 0)
    pltpu.sync_copy(d2v, d2_h.at[pl.ds(base, EW)])
    pltpu.sync_copy(idxv, idx_h.at[pl.ds(base, EW)])
    pltpu.sync_copy(tev, te_h.at[pl.ds(base, EW)])


def _pre(posx, posy, posz, an, jp, ip):
    k = pl.kernel(
        _pre_body,
        out_type=[jax.ShapeDtypeStruct((EPAD,), jnp.float32),
                  jax.ShapeDtypeStruct((EPAD,), jnp.int32),
                  jax.ShapeDtypeStruct((EPAD,), jnp.int32)],
        mesh=_sc_mesh(),
        scratch_types=[pltpu.VMEM((EW,), jnp.int32),
                       pltpu.VMEM((EW,), jnp.int32),
                       pltpu.VMEM((EW,), jnp.float32),
                       pltpu.VMEM((EW,), jnp.float32),
                       pltpu.VMEM((EW,), jnp.float32),
                       pltpu.VMEM((EW,), jnp.float32),
                       pltpu.VMEM((EW,), jnp.float32),
                       pltpu.VMEM((EW,), jnp.float32),
                       pltpu.VMEM((EW,), jnp.int32),
                       pltpu.VMEM((EW,), jnp.float32),
                       pltpu.VMEM((EW,), jnp.int32),
                       pltpu.VMEM((EW,), jnp.int32),
                       pltpu.SemaphoreType.DMA],
    )
    return k(posx, posy, posz, an, jp, ip)


def _gs_body(tab_h, filt_h, gidx_h, sidx_h, zeros_h, out_h,
             gidx_v, sidx_v, pa, pb, fa, fb, acc,
             sem_pa, sem_pb, sem_fa, sem_fb):
    cid = lax.axis_index("c")
    sid = lax.axis_index("s")
    w = sid * NC + cid
    base = w * NCHUNK
    r0 = sid * NSUB
    pltpu.sync_copy(zeros_h.at[pl.ds(r0, NSUB)], acc.at[pl.ds(r0, NSUB)])
    pltpu.sync_copy(gidx_h.at[pl.ds(w * EW, EW)], gidx_v)
    pltpu.sync_copy(sidx_h.at[pl.ds(base, NCHUNK)], sidx_v)
    plsc.subcore_barrier()

    def fire(c, pbuf, fbuf, psem, fsem):
        pltpu.async_copy(tab_h.at[gidx_v.at[pl.ds(c * CH, CH)]], pbuf, psem)
        pltpu.async_copy(filt_h.at[pl.ds((base + c) * CH, CH)], fbuf, fsem)

    def drain(c, pbuf, fbuf, psem, fsem):
        pltpu.make_async_copy(tab_h.at[gidx_v.at[pl.ds(0, CH)]], pbuf,
                              psem).wait()
        pltpu.make_async_copy(filt_h.at[pl.ds(base * CH, CH)], fbuf,
                              fsem).wait()

    def mul_scatter(c, pbuf, fbuf):
        def row(r, carry):
            for h in range(H // 16):
                sl = pl.ds(h * 16, 16)
                pbuf[r, sl] = pbuf[r, sl] * fbuf[r, sl]
            return carry

        lax.fori_loop(0, CH, row, 0)
        pltpu.sync_copy(pbuf, acc.at[sidx_v.at[c]], add=True)

    fire(0, pa, fa, sem_pa, sem_fa)

    def k_body(k, carry):
        c = 2 * k
        fire(c + 1, pb, fb, sem_pb, sem_fb)
        drain(c, pa, fa, sem_pa, sem_fa)
        mul_scatter(c, pa, fa)

        @pl.when(c + 2 < NCHUNK)
        def _():
            fire(c + 2, pa, fa, sem_pa, sem_fa)

        drain(c + 1, pb, fb, sem_pb, sem_fb)
        mul_scatter(c + 1, pb, fb)
        return carry

    lax.fori_loop(0, NCHUNK // 2, k_body, 0)
    plsc.subcore_barrier()
    pltpu.sync_copy(acc.at[pl.ds(r0, NSUB)], out_h.at[cid, pl.ds(r0, NSUB)])


def _gs(tab, filt, gidx2d, sidx2d, zerosN):
    buf = pltpu.VMEM((CH, H), jnp.float32)
    k = pl.kernel(
        _gs_body,
        out_type=jax.ShapeDtypeStruct((NC, NPAD, H), jnp.float32),
        mesh=_sc_mesh(),
        scratch_types=[pltpu.VMEM((EW,), jnp.int32),
                       pltpu.VMEM((NCHUNK, CH), jnp.int32),
                       buf, buf, buf, buf,
                       pltpu.VMEM_SHARED((NPAD, H), jnp.float32),
                       pltpu.SemaphoreType.DMA, pltpu.SemaphoreType.DMA,
                       pltpu.SemaphoreType.DMA, pltpu.SemaphoreType.DMA],
    )
    return k(tab, filt, gidx2d, sidx2d, zerosN)


def _phi0_kernel(an_ref, emb_ref, w10, b10, w20, b20, w11, b11, w21, b21,
                 out_ref):
    a = an_ref[0, 0, :]
    x = jnp.where(a[:, None] == 0, emb_ref[0:1, :], emb_ref[1:2, :])
    _phi_core(x, w10, b10, w20, b20, w11, b11, w21, b21, out_ref)


def _phi0(an3, embed, w10, b10, w20, b20, w11, b11, w21, b21):
    nb = N // BN
    wspec = pl.BlockSpec((H, H), lambda b: (0, 0))
    bspec = pl.BlockSpec((1, H), lambda b: (0, 0))
    return pl.pallas_call(
        _phi0_kernel,
        grid=(nb,),
        in_specs=[pl.BlockSpec((1, 1, BN), lambda b: (b, 0, 0)),
                  pl.BlockSpec((T, H), lambda b: (0, 0)),
                  wspec, bspec, wspec, bspec, wspec, bspec, wspec, bspec],
        out_specs=pl.BlockSpec((2, BN, H), lambda b: (0, b, 0)),
        out_shape=jax.ShapeDtypeStruct((2, N, H), jnp.float32),
    )(an3, embed, w10, b10, w20, b20, w11, b11, w21, b21)


def _phi_core(x, w10, b10, w20, b20, w11, b11, w21, b21, out_ref):
    h0 = jax.nn.silu(jnp.dot(x, w10[...], preferred_element_type=jnp.float32)
                     + b10[...])
    p0 = jnp.dot(h0, w20[...], preferred_element_type=jnp.float32) + b20[...]
    h1 = jax.nn.silu(jnp.dot(x, w11[...], preferred_element_type=jnp.float32)
                     + b11[...])
    p1 = jnp.dot(h1, w21[...], preferred_element_type=jnp.float32) + b21[...]
    out_ref[0, :, :] = p0
    out_ref[1, :, :] = p1


def _phi_kernel(xp_ref, w10, b10, w20, b20, w11, b11, w21, b21, out_ref):
    x = xp_ref[0, :, :] + xp_ref[1, :, :]
    _phi_core(x, w10, b10, w20, b20, w11, b11, w21, b21, out_ref)


def _phi(xp, w10, b10, w20, b20, w11, b11, w21, b21):
    nb = N // BN
    wspec = pl.BlockSpec((H, H), lambda b: (0, 0))
    bspec = pl.BlockSpec((1, H), lambda b: (0, 0))
    return pl.pallas_call(
        _phi_kernel,
        grid=(nb,),
        in_specs=[pl.BlockSpec((2, BN, H), lambda b: (0, b, 0)),
                  wspec, bspec, wspec, bspec, wspec, bspec, wspec, bspec],
        out_specs=pl.BlockSpec((2, BN, H), lambda b: (0, b, 0)),
        out_shape=jax.ShapeDtypeStruct((2, N, H), jnp.float32),
    )(xp, w10, b10, w20, b20, w11, b11, w21, b21)


def _filt_kernel(d2_ref, te_ref, wf0, bf0, wf1, bf1, out_ref):
    d2 = d2_ref[0, 0, :]
    d = jnp.sqrt(d2)
    d = jnp.where(d <= 1e-6, 1e-6, d)
    u = d * (1.0 / RC)
    u2 = u * u
    u4 = u2 * u2
    u5 = u4 * u
    env = 1.0 - 21.0 * u5 + 35.0 * u5 * u - 15.0 * u5 * u2
    env = jnp.where(u < 1.0, env, 0.0)
    offs = lax.broadcasted_iota(jnp.int32, (1, NRBF), 1).astype(jnp.float32) * (
        1.0 / (NRBF - 1))
    delta = 1.0 / (NRBF - 1)
    coeff = -0.5 / (delta * delta)
    diff = u[:, None] - offs
    rbf = jnp.exp(coeff * (diff * diff)) * env[:, None]
    f0 = jnp.dot(rbf, wf0[...], preferred_element_type=jnp.float32) + bf0[...]
    f1 = jnp.dot(rbf, wf1[...], preferred_element_type=jnp.float32) + bf1[...]
    t = te_ref[0, 0, :]
    out_ref[...] = jnp.where(t[:, None] == 0, f0, f1)


def _filt(d2r, ter, wf0, bf0, wf1, bf1):
    nbe = EPAD // BE
    wspec = pl.BlockSpec((NRBF, H), lambda b: (0, 0))
    bspec = pl.BlockSpec((1, H), lambda b: (0, 0))
    return pl.pallas_call(
        _filt_kernel,
        grid=(nbe,),
        in_specs=[pl.BlockSpec((1, 1, BE), lambda b: (b, 0, 0)),
                  pl.BlockSpec((1, 1, BE), lambda b: (b, 0, 0)),
                  wspec, bspec, wspec, bspec],
        out_specs=pl.BlockSpec((BE, H), lambda b: (b, 0)),
        out_shape=jax.ShapeDtypeStruct((EPAD, H), jnp.float32),
    )(d2r, ter, wf0, bf0, wf1, bf1)


def _out_kernel(xp_ref, w1_ref, b1_ref, w2_ref, b2_ref, batch_ref, out_ref):
    b = pl.program_id(0)
    x = xp_ref[0, :, :] + xp_ref[1, :, :]
    h = jax.nn.silu(jnp.dot(x, w1_ref[...], preferred_element_type=jnp.float32)
                    + b1_ref[...]) * (1.0 / 0.6)
    pa = jnp.dot(h, w2_ref[...], preferred_element_type=jnp.float32) + b2_ref[...]
    bt = batch_ref[0, 0, :]
    oh = (bt[:, None] == lax.broadcasted_iota(jnp.int32, (1, NG), 1)
          ).astype(jnp.float32)
    partial = lax.dot_general(pa, oh, (((0,), (0,)), ((), ())))

    @pl.when(b == 0)
    def _():
        out_ref[...] = partial

    @pl.when(b != 0)
    def _():
        out_ref[...] = out_ref[...] + partial


def _out_stage(xp, Wo1, bo1, Wo2, bo2, batch3):
    nb = N // BN
    out = pl.pallas_call(
        _out_kernel,
        grid=(nb,),
        in_specs=[
            pl.BlockSpec((2, BN, H), lambda b: (0, b, 0)),
            pl.BlockSpec((H, H // 2), lambda b: (0, 0)),
            pl.BlockSpec((1, H // 2), lambda b: (0, 0)),
            pl.BlockSpec((H // 2, 1), lambda b: (0, 0)),
            pl.BlockSpec((1, 1), lambda b: (0, 0)),
            pl.BlockSpec((1, 1, BN), lambda b: (b, 0, 0)),
        ],
        out_specs=pl.BlockSpec((1, NG), lambda b: (0, 0)),
        out_shape=jax.ShapeDtypeStruct((1, NG), jnp.float32),
    )(xp, Wo1, bo1.reshape(1, -1), Wo2, bo2.reshape(1, 1), batch3)
    return out.reshape(NG)


def kernel(pos, embed, Wm1, bm1, Wm2, bm2, Wf, bf, Wo1, bo1, Wo2, bo2,
           atomic_number, edge_index, batch):
    f32 = jnp.float32
    pos = pos.astype(f32)
    posx = pos[:, 0]
    posy = pos[:, 1]
    posz = pos[:, 2]
    an = atomic_number.astype(jnp.int32)
    j = edge_index[0].astype(jnp.int32)
    i = edge_index[1].astype(jnp.int32)
    pad = EPAD - E
    jp = jnp.concatenate([j, jnp.zeros((pad,), jnp.int32)])
    ip_g = jnp.concatenate([i, jnp.zeros((pad,), jnp.int32)])
    ip_s = jnp.concatenate([i, jnp.full((pad,), TRASH, jnp.int32)])

    d2, idx_phi, te = _pre(posx, posy, posz, an, jp, ip_g)

    an3 = an.reshape(N // BN, 1, BN)
    d2r = d2.reshape(EPAD // BE, 1, BE)
    ter = te.reshape(EPAD // BE, 1, BE)
    iscat = ip_s.reshape(EPAD // CH, CH)
    zerosN = jnp.zeros((NPAD, H), f32)
    batch3 = batch.astype(jnp.int32).reshape(N // BN, 1, BN)

    def mw(l):
        return (Wm1[l, 0], bm1[l, 0].reshape(1, H),
                Wm2[l, 0, :, :H], bm2[l, 0, :H].reshape(1, H),
                Wm1[l, 1], bm1[l, 1].reshape(1, H),
                Wm2[l, 1, :, :H], bm2[l, 1, :H].reshape(1, H))

    filt0, filt1 = [_filt(d2r, ter,
                          Wf[l, 0, :, :H], bf[l, 0, :H].reshape(1, H),
                          Wf[l, 1, :, :H], bf[l, 1, :H].reshape(1, H))
                    for l in range(L)]
    phitab = _phi0(an3, embed.astype(f32), *mw(0))
    xp = _gs(phitab.reshape(2 * N, H), filt0, idx_phi, iscat, zerosN)
    phitab = _phi(xp, *mw(1))
    xp = _gs(phitab.reshape(2 * N, H), filt1, idx_phi, iscat, zerosN)

    return _out_stage(xp, Wo1, bo1, Wo2, bo2, batch3)


# speedup vs baseline: 1.0772x; 1.0658x over previous
import functools

import jax
import jax.numpy as jnp
from jax import lax
from jax.experimental import pallas as pl
from jax.experimental.pallas import tpu as pltpu
from jax.experimental.pallas import tpu_sc as plsc

N = 10000
E = 160000
H = 128
NRBF = 128
L = 2
T = 2
RC = 5.0
NG = 16

NC = 2
NS = 16
NW = NC * NS
CH = 64
EPAD = 163840
EW = EPAD // NW
NCHUNK = EW // CH
CHP = 128
NPRE = EW // CHP
NPAD = 10240
NSUB = NPAD // NS
TRASH = N
BE = 2048
BN = 1000


def _sc_mesh():
    return plsc.VectorSubcoreMesh(core_axis_name="c", subcore_axis_name="s",
                                  num_cores=NC, num_subcores=NS)


def _pre_body(posx_h, posy_h, posz_h, an_h, j_h, i_h, d2_h, idx_h, te_h,
              jv, iv, xj, yj, zj, xi, yi, zi, tv, d2v, idxv, tev, sem):
    w = lax.axis_index("s") * NC + lax.axis_index("c")
    base = w * EW
    pltpu.sync_copy(j_h.at[pl.ds(base, EW)], jv)
    pltpu.sync_copy(i_h.at[pl.ds(base, EW)], iv)

    def chunk(c, carry):
        s = c * CHP
        jc = jv.at[pl.ds(s, CHP)]
        ic = iv.at[pl.ds(s, CHP)]
        ds = pl.ds(s, CHP)
        descs = [
            pltpu.async_copy(posx_h.at[jc], xj.at[ds], sem),
            pltpu.async_copy(posy_h.at[jc], yj.at[ds], sem),
            pltpu.async_copy(posz_h.at[jc], zj.at[ds], sem),
            pltpu.async_copy(posx_h.at[ic], xi.at[ds], sem),
            pltpu.async_copy(posy_h.at[ic], yi.at[ds], sem),
            pltpu.async_copy(posz_h.at[ic], zi.at[ds], sem),
            pltpu.async_copy(an_h.at[ic], tv.at[ds], sem),
        ]
        for d in descs:
            d.wait()
        return carry

    lax.fori_loop(0, NPRE, chunk, 0)

    def q_body(q, carry):
        s = q * 16
        sl = pl.ds(s, 16)
        dx = xj[sl] - xi[sl]
        dy = yj[sl] - yi[sl]
        dz = zj[sl] - zi[sl]
        d2v[sl] = dx * dx + dy * dy + dz * dz
        t16 = tv[sl]
        idxv[sl] = t16 * N + jv[sl]
        tev[sl] = t16
        return carry

    lax.fori_loop(0, EW // 16, q_body, 0)
    pltpu.sync_copy(d2v, d2_h.at[pl.ds(base, EW)])
    pltpu.sync_copy(idxv, idx_h.at[pl.ds(base, EW)])
    pltpu.sync_copy(tev, te_h.at[pl.ds(base, EW)])


def _pre(posx, posy, posz, an, jp, ip):
    k = pl.kernel(
        _pre_body,
        out_type=[jax.ShapeDtypeStruct((EPAD,), jnp.float32),
                  jax.ShapeDtypeStruct((EPAD,), jnp.int32),
                  jax.ShapeDtypeStruct((EPAD,), jnp.int32)],
        mesh=_sc_mesh(),
        scratch_types=[pltpu.VMEM((EW,), jnp.int32),
                       pltpu.VMEM((EW,), jnp.int32),
                       pltpu.VMEM((EW,), jnp.float32),
                       pltpu.VMEM((EW,), jnp.float32),
                       pltpu.VMEM((EW,), jnp.float32),
                       pltpu.VMEM((EW,), jnp.float32),
                       pltpu.VMEM((EW,), jnp.float32),
                       pltpu.VMEM((EW,), jnp.float32),
                       pltpu.VMEM((EW,), jnp.int32),
                       pltpu.VMEM((EW,), jnp.float32),
                       pltpu.VMEM((EW,), jnp.int32),
                       pltpu.VMEM((EW,), jnp.int32),
                       pltpu.SemaphoreType.DMA],
    )
    return k(posx, posy, posz, an, jp, ip)


def _gs_body(tab_h, filt_h, gidx_h, sidx_h, zeros_h, out_h,
             gidx_v, sidx_v, pa, pb, fa, fb, acc,
             sem_pa, sem_pb, sem_fa, sem_fb):
    cid = lax.axis_index("c")
    sid = lax.axis_index("s")
    w = sid * NC + cid
    base = w * NCHUNK
    r0 = sid * NSUB
    pltpu.sync_copy(zeros_h.at[pl.ds(r0, NSUB)], acc.at[pl.ds(r0, NSUB)])
    pltpu.sync_copy(gidx_h.at[pl.ds(w * EW, EW)], gidx_v)
    pltpu.sync_copy(sidx_h.at[pl.ds(base, NCHUNK)], sidx_v)
    plsc.subcore_barrier()

    def fire(c, pbuf, fbuf, psem, fsem):
        pltpu.async_copy(tab_h.at[gidx_v.at[pl.ds(c * CH, CH)]], pbuf, psem)
        pltpu.async_copy(filt_h.at[pl.ds((base + c) * CH, CH)], fbuf, fsem)

    def drain(c, pbuf, fbuf, psem, fsem):
        pltpu.make_async_copy(tab_h.at[gidx_v.at[pl.ds(0, CH)]], pbuf,
                              psem).wait()
        pltpu.make_async_copy(filt_h.at[pl.ds(base * CH, CH)], fbuf,
                              fsem).wait()

    def mul_scatter(c, pbuf, fbuf):
        def row(r, carry):
            for h in range(H // 16):
                sl = pl.ds(h * 16, 16)
                pbuf[r, sl] = pbuf[r, sl] * fbuf[r, sl]
            return carry

        lax.fori_loop(0, CH, row, 0)
        pltpu.sync_copy(pbuf, acc.at[sidx_v.at[c]], add=True)

    fire(0, pa, fa, sem_pa, sem_fa)

    def k_body(k, carry):
        c = 2 * k
        fire(c + 1, pb, fb, sem_pb, sem_fb)
        drain(c, pa, fa, sem_pa, sem_fa)
        mul_scatter(c, pa, fa)

        @pl.when(c + 2 < NCHUNK)
        def _():
            fire(c + 2, pa, fa, sem_pa, sem_fa)

        drain(c + 1, pb, fb, sem_pb, sem_fb)
        mul_scatter(c + 1, pb, fb)
        return carry

    lax.fori_loop(0, NCHUNK // 2, k_body, 0)
    plsc.subcore_barrier()
    pltpu.sync_copy(acc.at[pl.ds(r0, NSUB)], out_h.at[cid, pl.ds(r0, NSUB)])


def _gs(tab, filt, gidx2d, sidx2d, zerosN):
    buf = pltpu.VMEM((CH, H), jnp.float32)
    k = pl.kernel(
        _gs_body,
        out_type=jax.ShapeDtypeStruct((NC, NPAD, H), jnp.float32),
        mesh=_sc_mesh(),
        scratch_types=[pltpu.VMEM((EW,), jnp.int32),
                       pltpu.VMEM((NCHUNK, CH), jnp.int32),
                       buf, buf, buf, buf,
                       pltpu.VMEM_SHARED((NPAD, H), jnp.float32),
                       pltpu.SemaphoreType.DMA, pltpu.SemaphoreType.DMA,
                       pltpu.SemaphoreType.DMA, pltpu.SemaphoreType.DMA],
    )
    return k(tab, filt, gidx2d, sidx2d, zerosN)


def _x0_kernel(an_ref, emb_ref, out_ref):
    a = an_ref[0, 0, :]
    m = a[:, None] == 0
    x0 = jnp.where(m, emb_ref[0:1, :], emb_ref[1:2, :])
    out_ref[0, :, :] = x0
    out_ref[1, :, :] = jnp.zeros_like(x0)


def _x0(an3, embed):
    nb = N // BN
    return pl.pallas_call(
        _x0_kernel,
        grid=(nb,),
        in_specs=[pl.BlockSpec((1, 1, BN), lambda b: (b, 0, 0)),
                  pl.BlockSpec((T, H), lambda b: (0, 0))],
        out_specs=pl.BlockSpec((2, BN, H), lambda b: (0, b, 0)),
        out_shape=jax.ShapeDtypeStruct((2, NPAD, H), jnp.float32),
    )(an3, embed)


def _phi_kernel(xp_ref, w10, b10, w20, b20, w11, b11, w21, b21, out_ref):
    x = xp_ref[0, :, :] + xp_ref[1, :, :]
    h0 = jax.nn.silu(jnp.dot(x, w10[...], preferred_element_type=jnp.float32)
                     + b10[...])
    p0 = jnp.dot(h0, w20[...], preferred_element_type=jnp.float32) + b20[...]
    h1 = jax.nn.silu(jnp.dot(x, w11[...], preferred_element_type=jnp.float32)
                     + b11[...])
    p1 = jnp.dot(h1, w21[...], preferred_element_type=jnp.float32) + b21[...]
    out_ref[0, :, :] = p0
    out_ref[1, :, :] = p1


def _phi(xp, w10, b10, w20, b20, w11, b11, w21, b21):
    nb = N // BN
    wspec = pl.BlockSpec((H, H), lambda b: (0, 0))
    bspec = pl.BlockSpec((1, H), lambda b: (0, 0))
    return pl.pallas_call(
        _phi_kernel,
        grid=(nb,),
        in_specs=[pl.BlockSpec((2, BN, H), lambda b: (0, b, 0)),
                  wspec, bspec, wspec, bspec, wspec, bspec, wspec, bspec],
        out_specs=pl.BlockSpec((2, BN, H), lambda b: (0, b, 0)),
        out_shape=jax.ShapeDtypeStruct((2, N, H), jnp.float32),
    )(xp, w10, b10, w20, b20, w11, b11, w21, b21)


def _filt_kernel(d2_ref, te_ref, wf0, bf0, wf1, bf1, out_ref):
    d2 = d2_ref[0, 0, :]
    d = jnp.sqrt(d2)
    d = jnp.where(d <= 1e-6, 1e-6, d)
    u = d * (1.0 / RC)
    u2 = u * u
    u4 = u2 * u2
    u5 = u4 * u
    env = 1.0 - 21.0 * u5 + 35.0 * u5 * u - 15.0 * u5 * u2
    env = jnp.where(u < 1.0, env, 0.0)
    offs = lax.broadcasted_iota(jnp.int32, (1, NRBF), 1).astype(jnp.float32) * (
        1.0 / (NRBF - 1))
    delta = 1.0 / (NRBF - 1)
    coeff = -0.5 / (delta * delta)
    diff = u[:, None] - offs
    rbf = jnp.exp(coeff * (diff * diff)) * env[:, None]
    f0 = jnp.dot(rbf, wf0[...], preferred_element_type=jnp.float32) + bf0[...]
    f1 = jnp.dot(rbf, wf1[...], preferred_element_type=jnp.float32) + bf1[...]
    t = te_ref[0, 0, :]
    out_ref[...] = jnp.where(t[:, None] == 0, f0, f1)


def _filt(d2r, ter, wf0, bf0, wf1, bf1):
    nbe = EPAD // BE
    wspec = pl.BlockSpec((NRBF, H), lambda b: (0, 0))
    bspec = pl.BlockSpec((1, H), lambda b: (0, 0))
    return pl.pallas_call(
        _filt_kernel,
        grid=(nbe,),
        in_specs=[pl.BlockSpec((1, 1, BE), lambda b: (b, 0, 0)),
                  pl.BlockSpec((1, 1, BE), lambda b: (b, 0, 0)),
                  wspec, bspec, wspec, bspec],
        out_specs=pl.BlockSpec((BE, H), lambda b: (b, 0)),
        out_shape=jax.ShapeDtypeStruct((EPAD, H), jnp.float32),
    )(d2r, ter, wf0, bf0, wf1, bf1)


def _out_kernel(xp_ref, w1_ref, b1_ref, w2_ref, b2_ref, batch_ref, out_ref):
    b = pl.program_id(0)
    x = xp_ref[0, :, :] + xp_ref[1, :, :]
    h = jax.nn.silu(jnp.dot(x, w1_ref[...], preferred_element_type=jnp.float32)
                    + b1_ref[...]) * (1.0 / 0.6)
    pa = jnp.dot(h, w2_ref[...], preferred_element_type=jnp.float32) + b2_ref[...]
    bt = batch_ref[0, 0, :]
    oh = (bt[:, None] == lax.broadcasted_iota(jnp.int32, (1, NG), 1)
          ).astype(jnp.float32)
    partial = lax.dot_general(pa, oh, (((0,), (0,)), ((), ())))

    @pl.when(b == 0)
    def _():
        out_ref[...] = partial

    @pl.when(b != 0)
    def _():
        out_ref[...] = out_ref[...] + partial


def _out_stage(xp, Wo1, bo1, Wo2, bo2, batch3):
    nb = N // BN
    out = pl.pallas_call(
        _out_kernel,
        grid=(nb,),
        in_specs=[
            pl.BlockSpec((2, BN, H), lambda b: (0, b, 0)),
            pl.BlockSpec((H, H // 2), lambda b: (0, 0)),
            pl.BlockSpec((1, H // 2), lambda b: (0, 0)),
            pl.BlockSpec((H // 2, 1), lambda b: (0, 0)),
            pl.BlockSpec((1, 1), lambda b: (0, 0)),
            pl.BlockSpec((1, 1, BN), lambda b: (b, 0, 0)),
        ],
        out_specs=pl.BlockSpec((1, NG), lambda b: (0, 0)),
        out_shape=jax.ShapeDtypeStruct((1, NG), jnp.float32),
    )(xp, Wo1, bo1.reshape(1, -1), Wo2, bo2.reshape(1, 1), batch3)
    return out.reshape(NG)


def kernel(pos, embed, Wm1, bm1, Wm2, bm2, Wf, bf, Wo1, bo1, Wo2, bo2,
           atomic_number, edge_index, batch):
    f32 = jnp.float32
    pos = pos.astype(f32)
    posx = pos[:, 0]
    posy = pos[:, 1]
    posz = pos[:, 2]
    an = atomic_number.astype(jnp.int32)
    j = edge_index[0].astype(jnp.int32)
    i = edge_index[1].astype(jnp.int32)
    pad = EPAD - E
    jp = jnp.concatenate([j, jnp.zeros((pad,), jnp.int32)])
    ip_g = jnp.concatenate([i, jnp.zeros((pad,), jnp.int32)])
    ip_s = jnp.concatenate([i, jnp.full((pad,), TRASH, jnp.int32)])

    d2, idx_phi, te = _pre(posx, posy, posz, an, jp, ip_g)

    an3 = an.reshape(N // BN, 1, BN)
    xp = _x0(an3, embed.astype(f32))

    d2r = d2.reshape(EPAD // BE, 1, BE)
    ter = te.reshape(EPAD // BE, 1, BE)
    iscat = ip_s.reshape(EPAD // CH, CH)
    zerosN = jnp.zeros((NPAD, H), f32)
    batch3 = batch.astype(jnp.int32).reshape(N // BN, 1, BN)

    filts = [_filt(d2r, ter,
                   Wf[l, 0, :, :H], bf[l, 0, :H].reshape(1, H),
                   Wf[l, 1, :, :H], bf[l, 1, :H].reshape(1, H))
             for l in range(L)]

    for l in range(L):
        phitab = _phi(xp,
                      Wm1[l, 0], bm1[l, 0].reshape(1, H),
                      Wm2[l, 0, :, :H], bm2[l, 0, :H].reshape(1, H),
                      Wm1[l, 1], bm1[l, 1].reshape(1, H),
                      Wm2[l, 1, :, :H], bm2[l, 1, :H].reshape(1, H))
        xp = _gs(phitab.reshape(2 * N, H), filts[l], idx_phi, iscat, zerosN)

    return _out_stage(xp, Wo1, bo1, Wo2, bo2, batch3)
